# Initial kernel scaffold; baseline (speedup 1.0000x reference)
import jax
import jax.numpy as jnp
from jax.experimental import pallas as pl


def kernel(embed_id, weight):
    raise NotImplementedError("write your pallas kernel here")


# speedup vs baseline: 3.7135x; 3.7135x over previous
import functools

import jax
import jax.numpy as jnp
from jax import lax
from jax.experimental import pallas as pl
from jax.experimental.pallas import tpu as pltpu
from jax.experimental.pallas import tpu_sc as plsc

_NUM_CORES = 2
_NUM_SUBCORES = 16
_NW = _NUM_CORES * _NUM_SUBCORES
_CHUNK = 128


@functools.lru_cache(maxsize=None)
def _make_gather(B, V, D):
    b_per_w = B // _NW
    n_chunks = b_per_w // _CHUNK
    mesh = plsc.VectorSubcoreMesh(core_axis_name="c", subcore_axis_name="s")

    @functools.partial(
        pl.kernel,
        out_type=jax.ShapeDtypeStruct((B, D), jnp.float32),
        mesh=mesh,
        scratch_types=[
            pltpu.VMEM((b_per_w,), jnp.int32),
            pltpu.VMEM((2, _CHUNK, D), jnp.float32),
            pltpu.SemaphoreType.DMA,
        ],
    )
    def gather_kernel(idx_hbm, table_hbm, out_hbm, idx_v, rows_v, gsem):
        wid = lax.axis_index("s") * _NUM_CORES + lax.axis_index("c")
        base = wid * b_per_w
        pltpu.sync_copy(idx_hbm.at[pl.ds(base, b_per_w)], idx_v)

        def body(c, _):
            off = c * _CHUNK
            buf = rows_v.at[c % 2]
            pltpu.async_copy(
                table_hbm.at[idx_v.at[pl.ds(off, _CHUNK)]], buf, gsem
            ).wait()
            pltpu.sync_copy(buf, out_hbm.at[pl.ds(base + off, _CHUNK)])
            return 0

        lax.fori_loop(0, n_chunks, body, 0)

    return gather_kernel


def kernel(embed_id, weight):
    V, D = weight.shape
    B = embed_id.size
    idx = embed_id.reshape(-1).astype(jnp.int32)
    out = _make_gather(B, V, D)(idx, weight)
    return out.reshape(embed_id.shape + (D,))


# speedup vs baseline: 4.0917x; 1.1019x over previous
import functools

import jax
import jax.numpy as jnp
from jax import lax
from jax.experimental import pallas as pl
from jax.experimental.pallas import tpu as pltpu
from jax.experimental.pallas import tpu_sc as plsc

_NUM_CORES = 2
_NUM_SUBCORES = 16
_NW = _NUM_CORES * _NUM_SUBCORES
_CHUNK = 128
_NBUF = 2


@functools.lru_cache(maxsize=None)
def _make_gather(B, V, D):
    b_per_w = B // _NW
    n_chunks = b_per_w // _CHUNK
    mesh = plsc.VectorSubcoreMesh(core_axis_name="c", subcore_axis_name="s")

    @functools.partial(
        pl.kernel,
        out_type=jax.ShapeDtypeStruct((B, D), jnp.float32),
        mesh=mesh,
        scratch_types=[
            pltpu.VMEM((b_per_w,), jnp.int32),
            pltpu.VMEM((_NBUF, _CHUNK, D), jnp.float32),
            pltpu.SemaphoreType.DMA,
            pltpu.SemaphoreType.DMA,
            pltpu.SemaphoreType.DMA,
            pltpu.SemaphoreType.DMA,
        ],
    )
    def gather_kernel(idx_hbm, table_hbm, out_hbm, idx_v, rows_v, g0, g1, o0, o1):
        gsem = (g0, g1)
        osem = (o0, o1)
        wid = lax.axis_index("s") * _NUM_CORES + lax.axis_index("c")
        base = wid * b_per_w
        pltpu.sync_copy(idx_hbm.at[pl.ds(base, b_per_w)], idx_v)

        def start_gather(c):
            b = c % _NBUF
            return pltpu.async_copy(
                table_hbm.at[idx_v.at[pl.ds(c * _CHUNK, _CHUNK)]],
                rows_v.at[b],
                gsem[b],
            )

        def start_store(c):
            b = c % _NBUF
            return pltpu.async_copy(
                rows_v.at[b],
                out_hbm.at[pl.ds(base + c * _CHUNK, _CHUNK)],
                osem[b],
            )

        g_d = {}
        o_d = {}
        for c in range(min(_NBUF, n_chunks)):
            g_d[c] = start_gather(c)
        for c in range(n_chunks):
            g_d[c].wait()
            o_d[c] = start_store(c)
            nxt = c + _NBUF
            if nxt < n_chunks:
                o_d[c].wait()
                g_d[nxt] = start_gather(nxt)
        for c in range(max(0, n_chunks - _NBUF), n_chunks):
            o_d[c].wait()

    return gather_kernel


def kernel(embed_id, weight):
    V, D = weight.shape
    B = embed_id.size
    idx = embed_id.reshape(-1).astype(jnp.int32)
    out = _make_gather(B, V, D)(idx, weight)
    return out.reshape(embed_id.shape + (D,))


# speedup vs baseline: 4.1278x; 1.0088x over previous
import functools

import jax
import jax.numpy as jnp
from jax import lax
from jax.experimental import pallas as pl
from jax.experimental.pallas import tpu as pltpu
from jax.experimental.pallas import tpu_sc as plsc

_NUM_CORES = 2
_NUM_SUBCORES = 16
_NW = _NUM_CORES * _NUM_SUBCORES
_CHUNK = 128
_NBUF = 3


@functools.lru_cache(maxsize=None)
def _make_gather(B, V, D):
    b_per_w = B // _NW
    n_chunks = b_per_w // _CHUNK
    mesh = plsc.VectorSubcoreMesh(core_axis_name="c", subcore_axis_name="s")

    @functools.partial(
        pl.kernel,
        out_type=jax.ShapeDtypeStruct((B, D), jnp.float32),
        mesh=mesh,
        scratch_types=[
            pltpu.VMEM((b_per_w,), jnp.int32),
            pltpu.VMEM((_NBUF, _CHUNK, D), jnp.float32),
            pltpu.SemaphoreType.DMA,
            pltpu.SemaphoreType.DMA,
            pltpu.SemaphoreType.DMA,
            pltpu.SemaphoreType.DMA,
            pltpu.SemaphoreType.DMA,
            pltpu.SemaphoreType.DMA,
        ],
    )
    def gather_kernel(
        idx_hbm, table_hbm, out_hbm, idx_v, rows_v, g0, g1, g2, o0, o1, o2
    ):
        gsem = (g0, g1, g2)
        osem = (o0, o1, o2)
        wid = lax.axis_index("s") * _NUM_CORES + lax.axis_index("c")
        base = wid * b_per_w
        pltpu.sync_copy(idx_hbm.at[pl.ds(base, b_per_w)], idx_v)

        def start_gather(c):
            b = c % _NBUF
            return pltpu.async_copy(
                table_hbm.at[idx_v.at[pl.ds(c * _CHUNK, _CHUNK)]],
                rows_v.at[b],
                gsem[b],
            )

        def start_store(c):
            b = c % _NBUF
            return pltpu.async_copy(
                rows_v.at[b],
                out_hbm.at[pl.ds(base + c * _CHUNK, _CHUNK)],
                osem[b],
            )

        g_d = {}
        o_d = {}
        pending = []
        for c in range(min(_NBUF, n_chunks)):
            g_d[c] = start_gather(c)
        for c in range(n_chunks):
            if c >= 1 and c - 1 + _NBUF < n_chunks:
                o_d[c - 1].wait()
                pending.remove(c - 1)
                g_d[c - 1 + _NBUF] = start_gather(c - 1 + _NBUF)
            g_d[c].wait()
            o_d[c] = start_store(c)
            pending.append(c)
        for c in pending:
            o_d[c].wait()

    return gather_kernel


def kernel(embed_id, weight):
    V, D = weight.shape
    B = embed_id.size
    idx = embed_id.reshape(-1).astype(jnp.int32)
    out = _make_gather(B, V, D)(idx, weight)
    return out.reshape(embed_id.shape + (D,))
